# baseline (speedup 1.0000x reference)
import functools

import jax
import jax.numpy as jnp
from jax import lax
from jax.experimental import pallas as pl
from jax.experimental.pallas import tpu as pltpu
from jax.experimental.pallas import tpu_sc as plsc

N = 10000
D = 128
DA = 144
E = 320000
NC = 2
NS = 16
CHUNK = 128
NBUF = 2
GRP = 8
ROWS = 2560
ROWS_PER_TILE = ROWS // NS
N_PAD = 10112
STRIPE = N_PAD // NS

_sc_mesh = plsc.VectorSubcoreMesh(core_axis_name="c", subcore_axis_name="s",
                                  num_cores=NC, num_subcores=NS)


@functools.partial(
    pl.kernel,
    out_type=[jax.ShapeDtypeStruct((N_PAD, DA), jnp.float32),
              jax.ShapeDtypeStruct((N_PAD, DA), jnp.float32)],
    mesh=_sc_mesh,
    scratch_types=[
        [pltpu.VMEM((CHUNK,), jnp.int32) for _ in range(NBUF)],
        [pltpu.VMEM((1, CHUNK), jnp.int32) for _ in range(NBUF)],
        [pltpu.VMEM((CHUNK, DA), jnp.float32) for _ in range(NBUF)],
        pltpu.VMEM_SHARED((N_PAD, DA), jnp.float32),
        [pltpu.SemaphoreType.DMA for _ in range(NBUF)],
        [pltpu.SemaphoreType.DMA for _ in range(NBUF)],
    ],
    compiler_params=pltpu.CompilerParams(use_tc_tiling_on_sc=False),
)
def _sc_accumulate(xu_aug, xi_aug, src_ui, dst_ui, src_iu, dst_iu, zeros,
                   acc_item, acc_user, sidx, didx, rows, acc_sh,
                   gsems, ssems):
    c = lax.axis_index("c")
    s = lax.axis_index("s")

    pltpu.sync_copy(zeros.at[pl.ds(s * STRIPE, STRIPE)],
                    acc_sh.at[pl.ds(s * STRIPE, STRIPE)])
    plsc.subcore_barrier()

    def run_edges(src_hbm, dst_hbm, x_hbm):
        def body(i, carry):
            row_a = s * ROWS_PER_TILE + i * 2
            gds, sds = [None] * NBUF, [None] * NBUF
            for k in range(NBUF):
                row = row_a + k
                pltpu.sync_copy(src_hbm.at[pl.ds(row * CHUNK, CHUNK)],
                                sidx[k])
                pltpu.sync_copy(dst_hbm.at[pl.ds(row, 1)], didx[k])
                gds[k] = pltpu.async_copy(x_hbm.at[sidx[k]], rows[k],
                                          gsems[k])
            for k in range(NBUF):
                gds[k].wait()
                pltpu.sync_copy(rows[k], acc_sh.at[didx[k].at[0]], add=True)
            return carry
        lax.fori_loop(0, ROWS_PER_TILE // NBUF, body, 0)

    @pl.when(c == 0)
    def _():
        run_edges(src_ui, dst_ui, xu_aug)

    @pl.when(c == 1)
    def _():
        run_edges(src_iu, dst_iu, xi_aug)

    plsc.subcore_barrier()

    @pl.when(c == 0)
    def _():
        pltpu.sync_copy(acc_sh.at[pl.ds(s * STRIPE, STRIPE)],
                        acc_item.at[pl.ds(s * STRIPE, STRIPE)])

    @pl.when(c == 1)
    def _():
        pltpu.sync_copy(acc_sh.at[pl.ds(s * STRIPE, STRIPE)],
                        acc_user.at[pl.ds(s * STRIPE, STRIPE)])


def _tc_body(acc_u, acc_i, xu, xi, wl_iu, wr_iu, b_iu, wl_ui, wr_ui, b_ui,
             out):
    for t, (acc, xd, wl, wr, b) in enumerate((
            (acc_u, xu, wl_iu, wr_iu, b_iu),
            (acc_i, xi, wl_ui, wr_ui, b_ui))):
        summed = acc[:N, :D]
        cnt = acc[:N, D:D + 1]
        mean = summed / jnp.maximum(cnt, 1.0)
        out[t] = (jnp.dot(mean, wl[...], preferred_element_type=jnp.float32)
                  + jnp.dot(xd[...], wr[...], preferred_element_type=jnp.float32)
                  + b[...])


def kernel(x_user, x_item, edge_index_rates, edge_index_rev,
           W_l_ui, W_r_ui, b_ui, W_l_iu, W_r_iu, b_iu):
    ones16 = jnp.ones((N, DA - D), jnp.float32)
    xu_aug = jnp.concatenate([x_user, ones16], axis=1)
    xi_aug = jnp.concatenate([x_item, ones16], axis=1)

    pad = ROWS * CHUNK - E

    def pad_edges(ei):
        src = jnp.concatenate([ei[0].astype(jnp.int32),
                               jnp.zeros((pad,), jnp.int32)])
        dst = jnp.concatenate([ei[1].astype(jnp.int32),
                               jnp.full((pad,), N, jnp.int32)])
        return src, dst.reshape(ROWS, CHUNK)

    src_ui, dst_ui = pad_edges(edge_index_rates)
    src_iu, dst_iu = pad_edges(edge_index_rev)
    zeros = jnp.zeros((N_PAD, DA), jnp.float32)

    acc_item, acc_user = _sc_accumulate(xu_aug, xi_aug, src_ui, dst_ui,
                                        src_iu, dst_iu, zeros)

    out = pl.pallas_call(
        _tc_body,
        out_shape=jax.ShapeDtypeStruct((2, N, D), jnp.float32),
    )(acc_user, acc_item, x_user, x_item,
      W_l_iu, W_r_iu, b_iu.reshape(1, D),
      W_l_ui, W_r_ui, b_ui.reshape(1, D))
    return out


# speedup vs baseline: 1.6824x; 1.6824x over previous
import functools

import jax
import jax.numpy as jnp
from jax import lax
from jax.experimental import pallas as pl
from jax.experimental.pallas import tpu as pltpu
from jax.experimental.pallas import tpu_sc as plsc

N = 10000
D = 128
DA = 144
E = 320000
NC = 2
NS = 16
CHUNK = 128
NBUF = 2
ROWS = 2512
ROWS_PER_TILE = ROWS // NS
N_PAD = 10112
STRIPE = N_PAD // NS

_sc_mesh = plsc.VectorSubcoreMesh(core_axis_name="c", subcore_axis_name="s",
                                  num_cores=NC, num_subcores=NS)


@functools.partial(
    pl.kernel,
    out_type=[jax.ShapeDtypeStruct((N_PAD, DA), jnp.float32),
              jax.ShapeDtypeStruct((N_PAD, DA), jnp.float32)],
    mesh=_sc_mesh,
    scratch_types=[
        [pltpu.VMEM((CHUNK,), jnp.int32) for _ in range(NBUF)],
        [pltpu.VMEM((1, CHUNK), jnp.int32) for _ in range(NBUF)],
        [pltpu.VMEM((CHUNK, DA), jnp.float32) for _ in range(NBUF)],
        pltpu.VMEM_SHARED((N_PAD, DA), jnp.float32),
        [pltpu.SemaphoreType.DMA for _ in range(NBUF)],
    ],
    compiler_params=pltpu.CompilerParams(use_tc_tiling_on_sc=False),
)
def _sc_accumulate(xu_aug, xi_aug, src_ui, dst_ui, src_iu, dst_iu, zeros,
                   acc_item, acc_user, sidx, didx, rows, acc_sh, gsems):
    c = lax.axis_index("c")
    s = lax.axis_index("s")

    pltpu.sync_copy(zeros.at[pl.ds(s * STRIPE, STRIPE)],
                    acc_sh.at[pl.ds(s * STRIPE, STRIPE)])
    plsc.subcore_barrier()

    def run_edges(src_hbm, dst_hbm, x_hbm):
        def chunk(row, k, desc):
            pltpu.sync_copy(src_hbm.at[pl.ds(row * CHUNK, CHUNK)], sidx[k])
            pltpu.sync_copy(dst_hbm.at[pl.ds(row, 1)], didx[k])
            gd = pltpu.async_copy(x_hbm.at[sidx[k]], rows[k], gsems[k])
            if desc is not None:
                kprev, gprev = desc
                gprev.wait()
                pltpu.sync_copy(rows[kprev], acc_sh.at[didx[kprev].at[0]],
                                add=True)
            return (k, gd)

        def body(i, carry):
            row_a = s * ROWS_PER_TILE + i * 2
            da = chunk(row_a, 0, None)
            db = chunk(row_a + 1, 1, da)
            _, gb = db
            gb.wait()
            pltpu.sync_copy(rows[1], acc_sh.at[didx[1].at[0]], add=True)
            return carry
        lax.fori_loop(0, ROWS_PER_TILE // 2, body, 0)
        row = s * ROWS_PER_TILE + ROWS_PER_TILE - 1
        _, gd = chunk(row, 0, None)
        gd.wait()
        pltpu.sync_copy(rows[0], acc_sh.at[didx[0].at[0]], add=True)

    @pl.when(c == 0)
    def _():
        run_edges(src_ui, dst_ui, xu_aug)

    @pl.when(c == 1)
    def _():
        run_edges(src_iu, dst_iu, xi_aug)

    plsc.subcore_barrier()

    @pl.when(c == 0)
    def _():
        pltpu.sync_copy(acc_sh.at[pl.ds(s * STRIPE, STRIPE)],
                        acc_item.at[pl.ds(s * STRIPE, STRIPE)])

    @pl.when(c == 1)
    def _():
        pltpu.sync_copy(acc_sh.at[pl.ds(s * STRIPE, STRIPE)],
                        acc_user.at[pl.ds(s * STRIPE, STRIPE)])


def _tc_body(acc_u, acc_i, xu, xi, wl_iu, wr_iu, b_iu, wl_ui, wr_ui, b_ui,
             out):
    for t, (acc, xd, wl, wr, b) in enumerate((
            (acc_u, xu, wl_iu, wr_iu, b_iu),
            (acc_i, xi, wl_ui, wr_ui, b_ui))):
        summed = acc[:N, :D]
        cnt = acc[:N, D:D + 1]
        mean = summed / jnp.maximum(cnt, 1.0)
        out[t] = (jnp.dot(mean, wl[...], preferred_element_type=jnp.float32)
                  + jnp.dot(xd[...], wr[...], preferred_element_type=jnp.float32)
                  + b[...])


def kernel(x_user, x_item, edge_index_rates, edge_index_rev,
           W_l_ui, W_r_ui, b_ui, W_l_iu, W_r_iu, b_iu):
    ones16 = jnp.ones((N, DA - D), jnp.float32)
    xu_aug = jnp.concatenate([x_user, ones16], axis=1)
    xi_aug = jnp.concatenate([x_item, ones16], axis=1)

    pad = ROWS * CHUNK - E

    def pad_edges(ei):
        src = jnp.concatenate([ei[0].astype(jnp.int32),
                               jnp.zeros((pad,), jnp.int32)])
        dst = jnp.concatenate([ei[1].astype(jnp.int32),
                               jnp.full((pad,), N, jnp.int32)])
        return src, dst.reshape(ROWS, CHUNK)

    src_ui, dst_ui = pad_edges(edge_index_rates)
    src_iu, dst_iu = pad_edges(edge_index_rev)
    zeros = jnp.zeros((N_PAD, DA), jnp.float32)

    acc_item, acc_user = _sc_accumulate(xu_aug, xi_aug, src_ui, dst_ui,
                                        src_iu, dst_iu, zeros)

    out = pl.pallas_call(
        _tc_body,
        out_shape=jax.ShapeDtypeStruct((2, N, D), jnp.float32),
    )(acc_user, acc_item, x_user, x_item,
      W_l_iu, W_r_iu, b_iu.reshape(1, D),
      W_l_ui, W_r_ui, b_ui.reshape(1, D))
    return out


# speedup vs baseline: 1.8546x; 1.1023x over previous
import functools

import jax
import jax.numpy as jnp
from jax import lax
from jax.experimental import pallas as pl
from jax.experimental.pallas import tpu as pltpu
from jax.experimental.pallas import tpu_sc as plsc

N = 10000
D = 128
DA = 144
E = 320000
NC = 2
NS = 16
CHUNK = 128
NBUF = 2
ROWS = 2512
ROWS_PER_TILE = ROWS // NS
N_PAD = 10112
STRIPE = N_PAD // NS

_sc_mesh = plsc.VectorSubcoreMesh(core_axis_name="c", subcore_axis_name="s",
                                  num_cores=NC, num_subcores=NS)


@functools.partial(
    pl.kernel,
    out_type=[jax.ShapeDtypeStruct((N_PAD, DA), jnp.float32),
              jax.ShapeDtypeStruct((N_PAD, DA), jnp.float32)],
    mesh=_sc_mesh,
    scratch_types=[
        [pltpu.VMEM((CHUNK,), jnp.int32) for _ in range(NBUF)],
        [pltpu.VMEM((1, CHUNK), jnp.int32) for _ in range(NBUF)],
        [pltpu.VMEM((CHUNK, DA), jnp.float32) for _ in range(NBUF)],
        pltpu.VMEM_SHARED((N_PAD, DA), jnp.float32),
        [pltpu.SemaphoreType.DMA for _ in range(NBUF)],
        pltpu.SemaphoreType.DMA,
    ],
    compiler_params=pltpu.CompilerParams(use_tc_tiling_on_sc=False),
)
def _sc_accumulate(xu_aug, xi_aug, src_ui, dst_ui, src_iu, dst_iu, zeros,
                   acc_item, acc_user, sidx, didx, rows, acc_sh, gsems,
                   ssem):
    c = lax.axis_index("c")
    s = lax.axis_index("s")

    pltpu.sync_copy(zeros.at[pl.ds(s * STRIPE, STRIPE)],
                    acc_sh.at[pl.ds(s * STRIPE, STRIPE)])
    plsc.subcore_barrier()

    def run_edges(src_hbm, dst_hbm, x_hbm):
        def chunk(row, k, desc):
            pltpu.sync_copy(src_hbm.at[pl.ds(row * CHUNK, CHUNK)], sidx[k])
            pltpu.sync_copy(dst_hbm.at[pl.ds(row, 1)], didx[k])
            gd = pltpu.async_copy(x_hbm.at[sidx[k]], rows[k], gsems[k])
            if desc is not None:
                kprev, gprev = desc
                gprev.wait()
                pltpu.sync_copy(rows[kprev], acc_sh.at[didx[kprev].at[0]],
                                add=True)
            return (k, gd)

        def drain_scatter():
            pltpu.make_async_copy(x_hbm.at[pl.ds(0, CHUNK)], rows[1],
                                  ssem).wait()

        def body(i, carry):
            row_a = s * ROWS_PER_TILE + i * 2
            da = chunk(row_a, 0, None)

            @pl.when(i > 0)
            def _():
                drain_scatter()

            db = chunk(row_a + 1, 1, da)
            _, gb = db
            gb.wait()
            pltpu.async_copy(rows[1], acc_sh.at[didx[1].at[0]], ssem,
                             add=True)
            return carry
        lax.fori_loop(0, ROWS_PER_TILE // 2, body, 0)
        row = s * ROWS_PER_TILE + ROWS_PER_TILE - 1
        pltpu.sync_copy(src_hbm.at[pl.ds(row * CHUNK, CHUNK)], sidx[0])
        pltpu.sync_copy(dst_hbm.at[pl.ds(row, 1)], didx[0])
        gd = pltpu.async_copy(x_hbm.at[sidx[0]], rows[0], gsems[0])
        drain_scatter()
        gd.wait()
        pltpu.sync_copy(rows[0], acc_sh.at[didx[0].at[0]], add=True)

    @pl.when(c == 0)
    def _():
        run_edges(src_ui, dst_ui, xu_aug)

    @pl.when(c == 1)
    def _():
        run_edges(src_iu, dst_iu, xi_aug)

    plsc.subcore_barrier()

    @pl.when(c == 0)
    def _():
        pltpu.sync_copy(acc_sh.at[pl.ds(s * STRIPE, STRIPE)],
                        acc_item.at[pl.ds(s * STRIPE, STRIPE)])

    @pl.when(c == 1)
    def _():
        pltpu.sync_copy(acc_sh.at[pl.ds(s * STRIPE, STRIPE)],
                        acc_user.at[pl.ds(s * STRIPE, STRIPE)])


def _tc_body(acc_u, acc_i, xu, xi, wl_iu, wr_iu, b_iu, wl_ui, wr_ui, b_ui,
             out):
    for t, (acc, xd, wl, wr, b) in enumerate((
            (acc_u, xu, wl_iu, wr_iu, b_iu),
            (acc_i, xi, wl_ui, wr_ui, b_ui))):
        summed = acc[:N, :D]
        cnt = acc[:N, D:D + 1]
        mean = summed / jnp.maximum(cnt, 1.0)
        out[t] = (jnp.dot(mean, wl[...], preferred_element_type=jnp.float32)
                  + jnp.dot(xd[...], wr[...], preferred_element_type=jnp.float32)
                  + b[...])


def kernel(x_user, x_item, edge_index_rates, edge_index_rev,
           W_l_ui, W_r_ui, b_ui, W_l_iu, W_r_iu, b_iu):
    ones16 = jnp.ones((N, DA - D), jnp.float32)
    xu_aug = jnp.concatenate([x_user, ones16], axis=1)
    xi_aug = jnp.concatenate([x_item, ones16], axis=1)

    pad = ROWS * CHUNK - E

    def pad_edges(ei):
        src = jnp.concatenate([ei[0].astype(jnp.int32),
                               jnp.zeros((pad,), jnp.int32)])
        dst = jnp.concatenate([ei[1].astype(jnp.int32),
                               jnp.full((pad,), N, jnp.int32)])
        return src, dst.reshape(ROWS, CHUNK)

    src_ui, dst_ui = pad_edges(edge_index_rates)
    src_iu, dst_iu = pad_edges(edge_index_rev)
    zeros = jnp.zeros((N_PAD, DA), jnp.float32)

    acc_item, acc_user = _sc_accumulate(xu_aug, xi_aug, src_ui, dst_ui,
                                        src_iu, dst_iu, zeros)

    out = pl.pallas_call(
        _tc_body,
        out_shape=jax.ShapeDtypeStruct((2, N, D), jnp.float32),
    )(acc_user, acc_item, x_user, x_item,
      W_l_iu, W_r_iu, b_iu.reshape(1, D),
      W_l_ui, W_r_ui, b_ui.reshape(1, D))
    return out
